# baseline (speedup 1.0000x reference)
import functools
import math

import jax
import jax.numpy as jnp
from jax import lax
from jax.experimental import pallas as pl
from jax.experimental.pallas import tpu as pltpu
from jax.experimental.pallas import tpu_sc as plsc

_NC = 2
_NS = 16
_NW = _NC * _NS
_L = 16

_B = 80
_RP = 640


def _proj_body(z_ref, wq_ref, bq_ref, wk_ref, bk_ref, wv_ref, bv_ref,
               q_ref, kv_ref):
    x = z_ref[...]
    dn = (((1,), (1,)), ((), ()))
    q_ref[...] = lax.dot_general(x, wq_ref[...], dn,
                                 preferred_element_type=jnp.float32) + bq_ref[...]
    kv_ref[:, :128] = lax.dot_general(x, wk_ref[...], dn,
                                      preferred_element_type=jnp.float32) + bk_ref[...]
    kv_ref[:, 128:] = lax.dot_general(x, wv_ref[...], dn,
                                      preferred_element_type=jnp.float32) + bv_ref[...]


def _project(z, Wq, bq, Wk, bk, Wv, bv):
    n, d = z.shape
    blk = 2000
    row = pl.BlockSpec((blk, d), lambda i: (i, 0))
    wsp = pl.BlockSpec((d, d), lambda i: (0, 0))
    bsp = pl.BlockSpec((1, d), lambda i: (0, 0))
    return pl.pallas_call(
        _proj_body,
        grid=(n // blk,),
        in_specs=[row, wsp, bsp, wsp, bsp, wsp, bsp],
        out_specs=[row, pl.BlockSpec((blk, 2 * d), lambda i: (i, 0))],
        out_shape=[jax.ShapeDtypeStruct((n, d), jnp.float32),
                   jax.ShapeDtypeStruct((n, 2 * d), jnp.float32)],
    )(z, Wq, bq.reshape(1, d), Wk, bk.reshape(1, d), Wv, bv.reshape(1, d))


def _edge_body(epw, chunks, npad, tau,
               kv_hbm, q_hbm, src_hbm, dst_hbm,
               hu_out, s_out, ex_out,
               src0, dst0, src1, dst1, kvbuf, qbuf, vbuf, wbuf,
               zsbuf, hu_sh, s_sh, gkq):
    c = lax.axis_index("c")
    sid = lax.axis_index("s")
    wid = sid * _NC + c
    z16 = jnp.zeros((_L,), jnp.float32)
    lane = lax.iota(jnp.int32, _L)
    sets = ((src0, dst0), (src1, dst1))

    def zrow(i, _):
        for d in range(8):
            vbuf[i, pl.ds(d * _L, _L)] = z16
        return 0
    lax.fori_loop(0, _B, zrow, 0)

    def zs(i, _):
        zsbuf[pl.ds(i * _L, _L)] = z16
        return 0
    lax.fori_loop(0, _RP // _L, zs, 0)

    row0 = sid * _RP
    for j in range(_RP // _B):
        pltpu.sync_copy(vbuf, hu_sh.at[pl.ds(row0 + j * _B, _B)])
    pltpu.sync_copy(zsbuf, s_sh.at[pl.ds(row0, _RP)])
    plsc.subcore_barrier()

    base = wid * epw

    def load_idx(bs, ci):
        src_v, dst_v = bs
        off = base + ci * _B
        pltpu.sync_copy(src_hbm.at[pl.ds(off, _B)], src_v)
        pltpu.sync_copy(dst_hbm.at[pl.ds(off, _B)], dst_v)

    def scatters(bs, ci):
        src_v, dst_v = bs
        off = base + ci * _B
        pltpu.sync_copy(vbuf, hu_sh.at[dst_v], add=True)
        pltpu.sync_copy(wbuf, s_sh.at[dst_v], add=True)
        pltpu.sync_copy(wbuf, ex_out.at[pl.ds(off, _B)])

    def compute():
        def grp(gi, _):
            i0 = pl.multiple_of(gi * _L, _L)

            def edge(j, dots):
                i = i0 + j
                acc = kvbuf[i, pl.ds(0, _L)] * qbuf[i, pl.ds(0, _L)]
                for d in range(1, 8):
                    acc = acc + kvbuf[i, pl.ds(d * _L, _L)] * qbuf[i, pl.ds(d * _L, _L)]
                for kk in (8, 4, 2, 1):
                    acc = acc + acc.at[jnp.bitwise_xor(lane, kk)].get(
                        mode="promise_in_bounds")
                return jnp.where(lane == j, acc, dots)
            dots = lax.fori_loop(0, _L, edge, z16, unroll=4)
            evv = jnp.exp(dots * tau)
            wbuf[pl.ds(i0, _L)] = evv
            return 0
        lax.fori_loop(0, _B // _L, grp, 0)

    def scale():
        def grp(gi, _):
            i0 = pl.multiple_of(gi * _L, _L)
            evv = wbuf[pl.ds(i0, _L)]

            def edge(j, _):
                i = i0 + j
                ev = evv.at[jnp.full((_L,), j, jnp.int32)].get(
                    mode="promise_in_bounds")
                for d in range(8):
                    vbuf[i, pl.ds(d * _L, _L)] = ev * kvbuf[i, pl.ds(128 + d * _L, _L)]
                return 0
            lax.fori_loop(0, _L, edge, 0, unroll=4)
            return 0
        lax.fori_loop(0, _B // _L, grp, 0)

    def gather_kq(bs, ci):
        src_v, dst_v = bs
        load_idx(bs, ci)
        return (pltpu.async_copy(kv_hbm.at[src_v], kvbuf, gkq),
                pltpu.async_copy(q_hbm.at[dst_v], qbuf, gkq))

    def half(bs, ci, dkq):
        dkq[0].wait()
        dkq[1].wait()
        compute()
        scale()

    def pair(g, _):
        ca = g * 2
        dkqa = gather_kq(sets[0], ca)
        half(sets[0], ca, dkqa)
        dkqb = gather_kq(sets[1], ca + 1)
        scatters(sets[0], ca)
        half(sets[1], ca + 1, dkqb)
        scatters(sets[1], ca + 1)
        return 0
    lax.fori_loop(0, chunks // 2, pair, 0)
    dkql = gather_kq(sets[0], chunks - 1)
    half(sets[0], chunks - 1, dkql)
    scatters(sets[0], chunks - 1)
    plsc.subcore_barrier()

    for j in range(_RP // _B):
        r = row0 + j * _B
        pltpu.sync_copy(hu_sh.at[pl.ds(r, _B)], vbuf)
        pltpu.sync_copy(vbuf, hu_out.at[c, pl.ds(r, _B)])
    pltpu.sync_copy(s_sh.at[pl.ds(row0, _RP)], zsbuf)
    pltpu.sync_copy(zsbuf, s_out.at[c, pl.ds(row0, _RP)])


@functools.lru_cache(maxsize=None)
def _edge_call(e, npad, tau):
    epw = e // _NW
    chunks = epw // _B
    mesh = plsc.VectorSubcoreMesh(core_axis_name="c", subcore_axis_name="s",
                                  num_cores=_NC, num_subcores=_NS)
    idx_t = pltpu.VMEM((_B,), jnp.int32)
    dma = pltpu.SemaphoreType.DMA
    return pl.kernel(
        functools.partial(_edge_body, epw, chunks, npad, tau),
        out_type=(jax.ShapeDtypeStruct((_NC, npad, 128), jnp.float32),
                  jax.ShapeDtypeStruct((_NC, npad), jnp.float32),
                  jax.ShapeDtypeStruct((e,), jnp.float32)),
        mesh=mesh,
        scratch_types=[
            idx_t, idx_t, idx_t, idx_t,
            pltpu.VMEM((_B, 256), jnp.float32),
            pltpu.VMEM((_B, 128), jnp.float32),
            pltpu.VMEM((_B, 128), jnp.float32),
            pltpu.VMEM((_B,), jnp.float32),
            pltpu.VMEM((_RP,), jnp.float32),
            pltpu.VMEM_SHARED((npad, 128), jnp.float32),
            pltpu.VMEM_SHARED((npad,), jnp.float32),
            dma,
        ],
    )


def _fin_body(hu_ref, s_ref, h_ref, st_ref):
    st = s_ref[0] + s_ref[1]
    st_ref[...] = st
    h_ref[...] = (hu_ref[0] + hu_ref[1]) / (st + 1e-20)


def _finalize(hu, s2):
    npad = hu.shape[1]
    return pl.pallas_call(
        _fin_body,
        out_shape=[jax.ShapeDtypeStruct((npad, 128), jnp.float32),
                   jax.ShapeDtypeStruct((npad, 1), jnp.float32)],
    )(hu, s2)


_CB = 2000


def _alpha_body(epw, ex_hbm, dst_hbm, st_hbm, a_out, dst_v, ex_v, sv, av, sem):
    c = lax.axis_index("c")
    sid = lax.axis_index("s")
    wid = sid * _NC + c
    base = wid * epw

    def chunk(ci, _):
        off = base + ci * _CB
        pltpu.sync_copy(dst_hbm.at[pl.ds(off, _CB)], dst_v)
        pltpu.sync_copy(ex_hbm.at[pl.ds(off, _CB)], ex_v)
        pltpu.async_copy(st_hbm.at[dst_v], sv, sem).wait()

        def grp(i, _):
            i0 = pl.multiple_of(i * _L, _L)
            av[pl.ds(i0, _L)] = ex_v[pl.ds(i0, _L)] / (sv[pl.ds(i0, _L)] + 1e-20)
            return 0
        lax.fori_loop(0, _CB // _L, grp, 0)
        pltpu.sync_copy(av, a_out.at[pl.ds(off, _CB)])
        return 0
    lax.fori_loop(0, epw // _CB, chunk, 0)


@functools.lru_cache(maxsize=None)
def _alpha_call(e, npad):
    epw = e // _NW
    mesh = plsc.VectorSubcoreMesh(core_axis_name="c", subcore_axis_name="s",
                                  num_cores=_NC, num_subcores=_NS)
    return pl.kernel(
        functools.partial(_alpha_body, epw),
        out_type=jax.ShapeDtypeStruct((e,), jnp.float32),
        mesh=mesh,
        scratch_types=[
            pltpu.VMEM((_CB,), jnp.int32),
            pltpu.VMEM((_CB,), jnp.float32),
            pltpu.VMEM((_CB,), jnp.float32),
            pltpu.VMEM((_CB,), jnp.float32),
            pltpu.SemaphoreType.DMA,
        ],
    )


def kernel(z, edge_index, Wq, bq, Wk, bk, Wv, bv, Wse1, bse1, Wse2, bse2):
    n, d = z.shape
    e = edge_index.shape[1]
    npad = _NS * _RP
    tau = 1.0 / math.sqrt(d)
    src = edge_index[0]
    dst = edge_index[1]

    q, kv = _project(z, Wq, bq, Wk, bk, Wv, bv)
    hu, s2, ex = _edge_call(e, npad, tau)(kv, q, src, dst)
    h_pad, st = _finalize(hu, s2.reshape(_NC, npad, 1))
    alpha = _alpha_call(e, npad)(ex, dst, st.reshape(npad))
    return h_pad[:n], alpha


# speedup vs baseline: 1.6539x; 1.6539x over previous
import functools
import math

import jax
import jax.numpy as jnp
from jax import lax
from jax.experimental import pallas as pl
from jax.experimental.pallas import tpu as pltpu
from jax.experimental.pallas import tpu_sc as plsc

_NC = 2
_NS = 16
_NW = _NC * _NS
_L = 16

_B = 80
_RP = 640


def _proj_body(z_ref, wq_ref, bq_ref, wk_ref, bk_ref, wv_ref, bv_ref,
               q_ref, k_ref, v_ref):
    x = z_ref[...]
    dn = (((1,), (1,)), ((), ()))
    q_ref[...] = lax.dot_general(x, wq_ref[...], dn,
                                 preferred_element_type=jnp.float32) + bq_ref[...]
    k_ref[...] = lax.dot_general(x, wk_ref[...], dn,
                                 preferred_element_type=jnp.float32) + bk_ref[...]
    v_ref[...] = lax.dot_general(x, wv_ref[...], dn,
                                 preferred_element_type=jnp.float32) + bv_ref[...]


def _project(z, Wq, bq, Wk, bk, Wv, bv):
    n, d = z.shape
    blk = 2000
    row = pl.BlockSpec((blk, d), lambda i: (i, 0))
    wsp = pl.BlockSpec((d, d), lambda i: (0, 0))
    bsp = pl.BlockSpec((1, d), lambda i: (0, 0))
    out = jax.ShapeDtypeStruct((n, d), jnp.float32)
    return pl.pallas_call(
        _proj_body,
        grid=(n // blk,),
        in_specs=[row, wsp, bsp, wsp, bsp, wsp, bsp],
        out_specs=[row, row, row],
        out_shape=[out, out, out],
    )(z, Wq, bq.reshape(1, d), Wk, bk.reshape(1, d), Wv, bv.reshape(1, d))


def _edge_body(epw, chunks, npad, tau,
               k_hbm, q_hbm, v_hbm, src_hbm, dst_hbm,
               hu_out, s_out, ex_out,
               src0, dst0, src1, dst1, kbuf, qbuf, vbuf0, vbuf1,
               wbuf0, wbuf1, zsbuf, hu_sh, s_sh, gkq, gv, ssem, lsem):
    c = lax.axis_index("c")
    sid = lax.axis_index("s")
    wid = sid * _NC + c
    z16 = jnp.zeros((_L,), jnp.float32)
    lane = lax.iota(jnp.int32, _L)
    sets = ((src0, dst0, vbuf0, wbuf0), (src1, dst1, vbuf1, wbuf1))
    vbuf = vbuf0

    def zrow(i, _):
        for d in range(8):
            vbuf[i, pl.ds(d * _L, _L)] = z16
        return 0
    lax.fori_loop(0, _B, zrow, 0)

    def zs(i, _):
        zsbuf[pl.ds(i * _L, _L)] = z16
        return 0
    lax.fori_loop(0, _RP // _L, zs, 0)

    row0 = sid * _RP
    for j in range(_RP // _B):
        pltpu.sync_copy(vbuf, hu_sh.at[pl.ds(row0 + j * _B, _B)])
    pltpu.sync_copy(zsbuf, s_sh.at[pl.ds(row0, _RP)])
    plsc.subcore_barrier()

    base = wid * epw

    def load_idx(bs, ci):
        off = base + ci * _B
        pltpu.sync_copy(src_hbm.at[pl.ds(off, _B)], bs[0])
        pltpu.sync_copy(dst_hbm.at[pl.ds(off, _B)], bs[1])

    def load_idx_async(bs, ci):
        off = base + ci * _B
        return (pltpu.async_copy(src_hbm.at[pl.ds(off, _B)], bs[0], lsem),
                pltpu.async_copy(dst_hbm.at[pl.ds(off, _B)], bs[1], lsem))

    def scatters(bs, ci):
        src_v, dst_v, vb, wb = bs
        off = base + ci * _B
        pltpu.sync_copy(vb, hu_sh.at[dst_v], add=True)
        pltpu.sync_copy(wb, s_sh.at[dst_v], add=True)
        pltpu.sync_copy(wb, ex_out.at[pl.ds(off, _B)])

    def compute(wb):
        def grp(gi, _):
            i0 = pl.multiple_of(gi * _L, _L)

            def edge(j, dots):
                i = i0 + j
                acc = kbuf[i, pl.ds(0, _L)] * qbuf[i, pl.ds(0, _L)]
                for d in range(1, 8):
                    acc = acc + kbuf[i, pl.ds(d * _L, _L)] * qbuf[i, pl.ds(d * _L, _L)]
                for kk in (8, 4, 2, 1):
                    acc = acc + acc.at[jnp.bitwise_xor(lane, kk)].get(
                        mode="promise_in_bounds")
                return jnp.where(lane == j, acc, dots)
            dots = lax.fori_loop(0, _L, edge, z16, unroll=4)
            evv = jnp.exp(dots * tau)
            wb[pl.ds(i0, _L)] = evv
            return 0
        lax.fori_loop(0, _B // _L, grp, 0)

    def scale(vb, wb):
        def grp(gi, _):
            i0 = pl.multiple_of(gi * _L, _L)
            evv = wb[pl.ds(i0, _L)]

            def edge(j, _):
                i = i0 + j
                ev = evv.at[jnp.full((_L,), j, jnp.int32)].get(
                    mode="promise_in_bounds")
                for d in range(8):
                    vb[i, pl.ds(d * _L, _L)] = ev * vb[i, pl.ds(d * _L, _L)]
                return 0
            lax.fori_loop(0, _L, edge, 0, unroll=4)
            return 0
        lax.fori_loop(0, _B // _L, grp, 0)

    def gather_kq(bs):
        return (pltpu.async_copy(k_hbm.at[bs[0]], kbuf, gkq),
                pltpu.async_copy(q_hbm.at[bs[1]], qbuf, gkq))

    def gather_v(bs):
        return pltpu.async_copy(v_hbm.at[bs[0]], bs[2], gv)

    def half(bs, dkq, dv):
        dkq[0].wait()
        dkq[1].wait()
        compute(bs[3])
        dv.wait()
        scale(bs[2], bs[3])

    def pair(g, _):
        ca = g * 2
        load_idx(sets[0], ca)
        dkqa = gather_kq(sets[0])
        dva = gather_v(sets[0])
        la = load_idx_async(sets[1], ca + 1)
        la[0].wait()
        la[1].wait()
        dvb = gather_v(sets[1])
        half(sets[0], dkqa, dva)
        dkqb = gather_kq(sets[1])
        scatters(sets[0], ca)
        half(sets[1], dkqb, dvb)
        scatters(sets[1], ca + 1)
        return 0
    lax.fori_loop(0, chunks // 2, pair, 0)
    load_idx(sets[0], chunks - 1)
    dkql = gather_kq(sets[0])
    dvl = gather_v(sets[0])
    half(sets[0], dkql, dvl)
    scatters(sets[0], chunks - 1)
    plsc.subcore_barrier()

    for j in range(_RP // _B):
        r = row0 + j * _B
        pltpu.sync_copy(hu_sh.at[pl.ds(r, _B)], vbuf)
        pltpu.sync_copy(vbuf, hu_out.at[c, pl.ds(r, _B)])
    pltpu.sync_copy(s_sh.at[pl.ds(row0, _RP)], zsbuf)
    pltpu.sync_copy(zsbuf, s_out.at[c, pl.ds(row0, _RP)])


@functools.lru_cache(maxsize=None)
def _edge_call(e, npad, tau):
    epw = e // _NW
    chunks = epw // _B
    mesh = plsc.VectorSubcoreMesh(core_axis_name="c", subcore_axis_name="s",
                                  num_cores=_NC, num_subcores=_NS)
    idx_t = pltpu.VMEM((_B,), jnp.int32)
    row_t = pltpu.VMEM((_B, 128), jnp.float32)
    dma = pltpu.SemaphoreType.DMA
    return pl.kernel(
        functools.partial(_edge_body, epw, chunks, npad, tau),
        out_type=(jax.ShapeDtypeStruct((_NC, npad, 128), jnp.float32),
                  jax.ShapeDtypeStruct((_NC, npad), jnp.float32),
                  jax.ShapeDtypeStruct((e,), jnp.float32)),
        mesh=mesh,
        scratch_types=[
            idx_t, idx_t, idx_t, idx_t,
            row_t, row_t,
            row_t, row_t,
            pltpu.VMEM((_B,), jnp.float32),
            pltpu.VMEM((_B,), jnp.float32),
            pltpu.VMEM((_RP,), jnp.float32),
            pltpu.VMEM_SHARED((npad, 128), jnp.float32),
            pltpu.VMEM_SHARED((npad,), jnp.float32),
            dma, dma, dma, dma,
        ],
    )


def _fin_body(hu_ref, s_ref, h_ref, st_ref):
    st = s_ref[0] + s_ref[1]
    st_ref[...] = st
    h_ref[...] = (hu_ref[0] + hu_ref[1]) / (st + 1e-20)


def _finalize(hu, s2):
    npad = hu.shape[1]
    return pl.pallas_call(
        _fin_body,
        out_shape=[jax.ShapeDtypeStruct((npad, 128), jnp.float32),
                   jax.ShapeDtypeStruct((npad, 1), jnp.float32)],
    )(hu, s2)


_CB = 2000


def _alpha_body(epw, ex_hbm, dst_hbm, st_hbm, a_out, dst_v, ex_v, sv, av, sem):
    c = lax.axis_index("c")
    sid = lax.axis_index("s")
    wid = sid * _NC + c
    base = wid * epw

    def chunk(ci, _):
        off = base + ci * _CB
        pltpu.sync_copy(dst_hbm.at[pl.ds(off, _CB)], dst_v)
        pltpu.sync_copy(ex_hbm.at[pl.ds(off, _CB)], ex_v)
        pltpu.async_copy(st_hbm.at[dst_v], sv, sem).wait()

        def grp(i, _):
            i0 = pl.multiple_of(i * _L, _L)
            av[pl.ds(i0, _L)] = ex_v[pl.ds(i0, _L)] / (sv[pl.ds(i0, _L)] + 1e-20)
            return 0
        lax.fori_loop(0, _CB // _L, grp, 0)
        pltpu.sync_copy(av, a_out.at[pl.ds(off, _CB)])
        return 0
    lax.fori_loop(0, epw // _CB, chunk, 0)


@functools.lru_cache(maxsize=None)
def _alpha_call(e, npad):
    epw = e // _NW
    mesh = plsc.VectorSubcoreMesh(core_axis_name="c", subcore_axis_name="s",
                                  num_cores=_NC, num_subcores=_NS)
    return pl.kernel(
        functools.partial(_alpha_body, epw),
        out_type=jax.ShapeDtypeStruct((e,), jnp.float32),
        mesh=mesh,
        scratch_types=[
            pltpu.VMEM((_CB,), jnp.int32),
            pltpu.VMEM((_CB,), jnp.float32),
            pltpu.VMEM((_CB,), jnp.float32),
            pltpu.VMEM((_CB,), jnp.float32),
            pltpu.SemaphoreType.DMA,
        ],
    )


def kernel(z, edge_index, Wq, bq, Wk, bk, Wv, bv, Wse1, bse1, Wse2, bse2):
    n, d = z.shape
    e = edge_index.shape[1]
    npad = _NS * _RP
    tau = 1.0 / math.sqrt(d)
    src = edge_index[0]
    dst = edge_index[1]

    q, k, v = _project(z, Wq, bq, Wk, bk, Wv, bv)
    hu, s2, ex = _edge_call(e, npad, tau)(k, q, v, src, dst)
    h_pad, st = _finalize(hu, s2.reshape(_NC, npad, 1))
    alpha = _alpha_call(e, npad)(ex, dst, st.reshape(npad))
    return h_pad[:n], alpha
